# baseline (speedup 1.0000x reference)
import functools

import jax
import jax.numpy as jnp
from jax import lax
from jax.experimental import pallas as pl
from jax.experimental.pallas import tpu as pltpu
from jax.experimental.pallas import tpu_sc as plsc

KNN = 16
RBLK = 512
CG = 8
GW = 32
NG = 4


def _score_body(xr_ref, xf_ref, s_ref):
    xr = xr_ref[0]
    xf = xf_ref[0]
    s = lax.dot_general(xr, xf, (((0,), (0,)), ((), ())),
                        preferred_element_type=jnp.float32)
    xx = jnp.sum(xf * xf, axis=0, keepdims=True)
    s_ref[0, 0] = 2.0 * s - xx


def _score_call(x):
    b, c, n = x.shape
    nblk = n // RBLK
    return pl.pallas_call(
        _score_body,
        grid=(b, nblk),
        in_specs=[
            pl.BlockSpec((1, c, RBLK), lambda i, j: (i, 0, j)),
            pl.BlockSpec((1, c, n), lambda i, j: (i, 0, 0)),
        ],
        out_specs=pl.BlockSpec((1, 1, RBLK, n), lambda i, j: (i, j, 0, 0)),
        out_shape=jax.ShapeDtypeStruct((b, nblk, RBLK, n), jnp.float32),
    )(x, x)


def _merge16(va, ia, vb, ib):
    vbr = lax.rev(vb, (0,))
    ibr = lax.rev(ib, (0,))
    take = va >= vbr
    vm = jnp.where(take, va, vbr)
    im = jnp.where(take, ia, ibr)
    return plsc.sort_key_val(vm, im)


def _topk_call(score):
    b, nblk, rblk, n = score.shape
    rows = b * nblk * rblk
    rpt = rows // 32
    rpb = 16
    nblks = rpt // rpb

    mesh = plsc.VectorSubcoreMesh(core_axis_name="c", subcore_axis_name="s")

    @functools.partial(
        pl.kernel,
        mesh=mesh,
        compiler_params=pltpu.CompilerParams(needs_layout_passes=False),
        out_type=jax.ShapeDtypeStruct((rows * KNN,), jnp.int32),
        scratch_types=[
            pltpu.VMEM((rpb, n), jnp.float32),
            pltpu.VMEM((rpb, n), jnp.float32),
            pltpu.VMEM((rpt * KNN,), jnp.int32),
            pltpu.SemaphoreType.DMA,
            pltpu.SemaphoreType.DMA,
        ],
    )
    def sc_topk(score_hbm, idx_hbm, buf0, buf1, idxout, s0, s1):
        wid = lax.axis_index("s") * 2 + lax.axis_index("c")
        row0 = wid * rpt
        bb = row0 // (nblk * rblk)
        jb = (row0 % (nblk * rblk)) // rblk
        rb0 = row0 % rblk
        lane = lax.iota(jnp.int32, 16)
        sems = (s0, s1)
        bufs = (buf0, buf1)

        def dma(blk, par):
            src = score_hbm.at[bb, jb, pl.ds(rb0 + blk * rpb, rpb), :]
            return pltpu.make_async_copy(src, bufs[par], sems[par])

        dma(0, 0).start()
        dma(1, 1).start()

        def do_row(r, blk, par):
            buf = bufs[par]
            pairs = []
            for t in range(8):
                cm = buf[r, pl.ds(16 * t, 16)]
                for j in range(1, 16):
                    cm = jnp.maximum(
                        cm, buf[r, pl.ds(128 * j + 16 * t, 16)])
                pairs.append(plsc.sort_key_val(cm, lane + 16 * t))
            while len(pairs) > 1:
                pairs = [_merge16(*pairs[i], *pairs[i + 1])
                         for i in range(0, len(pairs), 2)]
            ci = pairs[0][1]
            rvec = jnp.full((16,), r, jnp.int32)
            cands = []
            for j in range(16):
                g = plsc.load_gather(buf, [rvec, ci + 128 * j])
                cands.append(plsc.sort_key_val(g, ci + 128 * j))
            while len(cands) > 1:
                cands = [_merge16(*cands[i], *cands[i + 1])
                         for i in range(0, len(cands), 2)]
            gidx = cands[0][1]
            idxout[pl.ds((blk * rpb + r) * KNN, 16)] = gidx

        def do_block(sb, _):
            for par in range(2):
                blk = sb * 2 + par
                dma(blk, par).wait()

                def row2(r2, c):
                    do_row(r2 * 2, blk, par)
                    do_row(r2 * 2 + 1, blk, par)
                    return c
                lax.fori_loop(0, rpb // 2, row2, 0)
                nxt = blk + 2

                @pl.when(nxt < nblks)
                def _():
                    dma(nxt, par).start()
            return 0

        lax.fori_loop(0, nblks // 2, do_block, 0)
        pltpu.sync_copy(idxout, idx_hbm.at[pl.ds(row0 * KNN, rpt * KNN)])

    return sc_topk(score)


def _lin_body(x_ref, w_ref, y1_ref, y2_ref):
    xb = x_ref[0]
    wb = w_ref[...]
    c = xb.shape[0]
    n = xb.shape[1]
    n = xb.shape[1]
    q = n // 4
    w1 = wb[:, :c]
    w2m1 = wb[:, c:] - w1
    dn = (((0,), (1,)), ((), ()))
    y1 = lax.dot_general(xb, w1, dn, preferred_element_type=jnp.float32)
    y2 = lax.dot_general(xb, w2m1, dn, preferred_element_type=jnp.float32)
    for s in range(4):
        y1_ref[0, 0, :, pl.ds(s * GW, GW)] = y1[s * q:(s + 1) * q, :]
        y2_ref[0, 0, :, pl.ds(s * GW, GW)] = y2[s * q:(s + 1) * q, :]


def _lin_call(x, w):
    b, c, n = x.shape
    q = n // 4
    return pl.pallas_call(
        _lin_body,
        grid=(b, CG),
        in_specs=[
            pl.BlockSpec((1, c, n), lambda i, j: (i, 0, 0)),
            pl.BlockSpec((GW, 2 * c), lambda i, j: (j, 0)),
        ],
        out_specs=[
            pl.BlockSpec((1, 1, q, 128), lambda i, j: (i, j, 0, 0)),
            pl.BlockSpec((1, 1, q, 128), lambda i, j: (i, j, 0, 0)),
        ],
        out_shape=[
            jax.ShapeDtypeStruct((b, CG, q, 128), jnp.float32),
            jax.ShapeDtypeStruct((b, CG, q, 128), jnp.float32),
        ],
    )(x, w)


def _bcast_lane(v, k):
    dn = lax.GatherDimensionNumbers(
        offset_dims=(), collapsed_slice_dims=(0,), start_index_map=(0,))
    return lax.gather(v, jnp.full((16, 1), k, jnp.int32), dn, (1,),
                      mode=lax.GatherScatterMode.PROMISE_IN_BOUNDS)


def _gather_call(y1w, idxf, b, n):
    nq = n // 4
    jn = nq // NG
    mesh = plsc.VectorSubcoreMesh(core_axis_name="c", subcore_axis_name="s")

    @functools.partial(
        pl.kernel,
        mesh=mesh,
        compiler_params=pltpu.CompilerParams(needs_layout_passes=False),
        out_type=[
            jax.ShapeDtypeStruct((b, CG, nq, 128), jnp.float32),
            jax.ShapeDtypeStruct((b, CG, nq, 128), jnp.float32),
            jax.ShapeDtypeStruct((b, CG, nq, 128), jnp.float32),
        ],
        scratch_types=[
            pltpu.VMEM((nq, 128), jnp.float32),
            pltpu.VMEM((NG * jn * KNN,), jnp.int32),
            pltpu.VMEM((jn, 128), jnp.float32),
            pltpu.VMEM((jn, 128), jnp.float32),
            pltpu.VMEM((jn, 128), jnp.float32),
            pltpu.SemaphoreType.DMA,
            pltpu.SemaphoreType.DMA,
        ],
    )
    def sc_gather(y1_hbm, idx_hbm, outm_hbm, outs_hbm, outq_hbm,
                  y1v, idxv, mv, sv, qv, sin, sout):
        wid = lax.axis_index("s") * 2 + lax.axis_index("c")
        g = wid // NG
        ng = wid % NG
        lane = lax.iota(jnp.int32, 16)

        def outdrain(bp):
            pltpu.make_async_copy(
                mv, outm_hbm.at[bp, g, pl.ds(ng * jn, jn), :], sout).wait()
            pltpu.make_async_copy(
                sv, outs_hbm.at[bp, g, pl.ds(ng * jn, jn), :], sout).wait()
            pltpu.make_async_copy(
                qv, outq_hbm.at[bp, g, pl.ds(ng * jn, jn), :], sout).wait()

        def idma(bi, q4):
            return pltpu.make_async_copy(
                idx_hbm.at[pl.ds((bi * n + q4 * nq + ng * jn) * KNN,
                                 jn * KNN)],
                idxv.at[pl.ds(q4 * jn * KNN, jn * KNN)], sin)

        for bi in range(b):
            for q4 in range(4):
                idma(bi, q4).start()
            pltpu.sync_copy(y1_hbm.at[bi, g], y1v)
            for q4 in range(4):
                idma(bi, q4).wait()
            if bi > 0:
                outdrain(bi - 1)

            for q4 in range(4):
                def n_body(j, _, q4=q4):
                    irow = idxv[pl.ds((q4 * jn + j) * KNN, 16)]
                    rav = irow & 511
                    cav = (irow >> 9) << 5
                    vm0 = jnp.full((16,), -jnp.inf, jnp.float32)
                    vm1 = vm0
                    vs0 = jnp.zeros((16,), jnp.float32)
                    vs1 = vs0
                    vq0 = vs0
                    vq1 = vs0
                    for k in range(KNN):
                        rb_ = _bcast_lane(rav, k)
                        cb_ = _bcast_lane(cav, k) + lane
                        g0 = plsc.load_gather(y1v, [rb_, cb_])
                        g1 = plsc.load_gather(y1v, [rb_, cb_ + 16])
                        vm0 = jnp.maximum(vm0, g0)
                        vm1 = jnp.maximum(vm1, g1)
                        vs0 = vs0 + g0
                        vs1 = vs1 + g1
                        vq0 = vq0 + g0 * g0
                        vq1 = vq1 + g1 * g1
                    mv[j, pl.ds(q4 * GW, 16)] = vm0
                    mv[j, pl.ds(q4 * GW + 16, 16)] = vm1
                    sv[j, pl.ds(q4 * GW, 16)] = vs0
                    sv[j, pl.ds(q4 * GW + 16, 16)] = vs1
                    qv[j, pl.ds(q4 * GW, 16)] = vq0
                    qv[j, pl.ds(q4 * GW + 16, 16)] = vq1
                    return 0

                lax.fori_loop(0, jn, n_body, 0)
            pltpu.make_async_copy(
                mv, outm_hbm.at[bi, g, pl.ds(ng * jn, jn), :], sout).start()
            pltpu.make_async_copy(
                sv, outs_hbm.at[bi, g, pl.ds(ng * jn, jn), :], sout).start()
            pltpu.make_async_copy(
                qv, outq_hbm.at[bi, g, pl.ds(ng * jn, jn), :], sout).start()
        outdrain(b - 1)

    return sc_gather(y1w, idxf)


def _fold4(x):
    return (x[:, 0:GW] + x[:, GW:2 * GW] + x[:, 2 * GW:3 * GW]
            + x[:, 3 * GW:4 * GW])


def _stats_body(y2_ref, s1_ref, q1_ref, part_ref):
    y2b = y2_ref[0, 0]
    s1b = s1_ref[0, 0]
    q1b = q1_ref[0, 0]
    g1 = _fold4(jnp.sum(s1b, axis=0, keepdims=True))
    g2 = _fold4(jnp.sum(q1b, axis=0, keepdims=True))
    g3 = _fold4(jnp.sum(y2b * s1b, axis=0, keepdims=True))
    h1 = _fold4(jnp.sum(y2b, axis=0, keepdims=True))
    h2 = _fold4(jnp.sum(y2b * y2b, axis=0, keepdims=True))
    part_ref[0, 0] = jnp.concatenate([g1, g2, g3, h1, h2], axis=0)


def _stats_call(y2, s1, q1):
    b, cg, nq, _ = y2.shape
    return pl.pallas_call(
        _stats_body,
        grid=(b, cg),
        in_specs=[
            pl.BlockSpec((1, 1, nq, 128), lambda i, j: (i, j, 0, 0)),
            pl.BlockSpec((1, 1, nq, 128), lambda i, j: (i, j, 0, 0)),
            pl.BlockSpec((1, 1, nq, 128), lambda i, j: (i, j, 0, 0)),
        ],
        out_specs=pl.BlockSpec((1, 1, 5, GW), lambda i, j: (i, j, 0, 0)),
        out_shape=jax.ShapeDtypeStruct((b, cg, 5, GW), jnp.float32),
    )(y2, s1, q1)


def _final_body(m1_ref, y2_ref, part_ref, gam_ref, bet_ref, out_ref, *, m):
    sums = jnp.sum(part_ref[:, 0], axis=0)
    g1 = sums[0:1]
    g2 = sums[1:2]
    g3 = sums[2:3]
    h1 = sums[3:4]
    h2 = sums[4:5]
    kf = float(KNN)
    mean = (g1 + kf * h1) / m
    e2 = (g2 + 2.0 * g3 + kf * h2) / m
    var = e2 - mean * mean
    inv = lax.rsqrt(var + 1e-5)
    scale = gam_ref[0] * inv
    shift = bet_ref[0] - mean * scale
    scale4 = jnp.concatenate([scale] * 4, axis=1)
    shift4 = jnp.concatenate([shift] * 4, axis=1)
    v = m1_ref[0, 0] + y2_ref[0, 0]
    z = jnp.maximum(v * scale4 + shift4, 0.0)
    nq = z.shape[0]
    for q in range(4):
        out_ref[0, 0, :, pl.ds(q * nq, nq)] = z[:, q * GW:(q + 1) * GW].T


def _final_call(m1, y2, parts, gamma, beta):
    b, cg, nq, _ = m1.shape
    n = nq * 4
    m = float(b * n * KNN)
    return pl.pallas_call(
        functools.partial(_final_body, m=m),
        grid=(b, cg),
        in_specs=[
            pl.BlockSpec((1, 1, nq, 128), lambda i, j: (i, j, 0, 0)),
            pl.BlockSpec((1, 1, nq, 128), lambda i, j: (i, j, 0, 0)),
            pl.BlockSpec((b, 1, 5, GW), lambda i, j: (0, j, 0, 0)),
            pl.BlockSpec((1, 1, GW), lambda i, j: (j, 0, 0)),
            pl.BlockSpec((1, 1, GW), lambda i, j: (j, 0, 0)),
        ],
        out_specs=pl.BlockSpec((1, 1, GW, n), lambda i, j: (i, j, 0, 0)),
        out_shape=jax.ShapeDtypeStruct((b, cg, GW, n), jnp.float32),
    )(m1, y2, parts, gamma, beta)


def kernel(x, W, gamma, beta):
    b, c, n = x.shape
    o = W.shape[0]
    score = _score_call(x)
    y1, y2 = _lin_call(x, W)
    idxf = _topk_call(score)
    m1, s1, q1 = _gather_call(y1, idxf, b, n)
    parts = _stats_call(y2, s1, q1)
    out = _final_call(m1, y2, parts,
                      gamma.reshape(CG, 1, GW), beta.reshape(CG, 1, GW))
    return out.reshape(b, o, n)


# speedup vs baseline: 1.1799x; 1.1799x over previous
import functools

import jax
import jax.numpy as jnp
from jax import lax
from jax.experimental import pallas as pl
from jax.experimental.pallas import tpu as pltpu
from jax.experimental.pallas import tpu_sc as plsc

KNN = 16
RBLK = 512
CG = 8
GW = 32
NG = 4


def _score_body(xr_ref, xf_ref, s_ref):
    xr = xr_ref[0]
    xf = xf_ref[0]
    s = lax.dot_general(xr, xf, (((0,), (0,)), ((), ())),
                        preferred_element_type=jnp.float32)
    xx = jnp.sum(xf * xf, axis=0, keepdims=True)
    s_ref[0, 0] = 2.0 * s - xx


def _score_call(x):
    b, c, n = x.shape
    nblk = n // RBLK
    return pl.pallas_call(
        _score_body,
        grid=(b, nblk),
        in_specs=[
            pl.BlockSpec((1, c, RBLK), lambda i, j: (i, 0, j)),
            pl.BlockSpec((1, c, n), lambda i, j: (i, 0, 0)),
        ],
        out_specs=pl.BlockSpec((1, 1, RBLK, n), lambda i, j: (i, j, 0, 0)),
        out_shape=jax.ShapeDtypeStruct((b, nblk, RBLK, n), jnp.float32),
    )(x, x)


def _merge16(va, ia, vb, ib):
    vbr = lax.rev(vb, (0,))
    ibr = lax.rev(ib, (0,))
    take = va >= vbr
    vm = jnp.where(take, va, vbr)
    im = jnp.where(take, ia, ibr)
    return plsc.sort_key_val(vm, im)


def _topk_call(score):
    b, nblk, rblk, n = score.shape
    rows = b * nblk * rblk
    rpt = rows // 32
    rpb = 16
    nblks = rpt // rpb

    mesh = plsc.VectorSubcoreMesh(core_axis_name="c", subcore_axis_name="s")

    @functools.partial(
        pl.kernel,
        mesh=mesh,
        compiler_params=pltpu.CompilerParams(needs_layout_passes=False),
        out_type=jax.ShapeDtypeStruct((rows * KNN,), jnp.int32),
        scratch_types=[
            pltpu.VMEM((rpb, n), jnp.float32),
            pltpu.VMEM((rpb, n), jnp.float32),
            pltpu.VMEM((rpt * KNN,), jnp.int32),
            pltpu.SemaphoreType.DMA,
            pltpu.SemaphoreType.DMA,
        ],
    )
    def sc_topk(score_hbm, idx_hbm, buf0, buf1, idxout, s0, s1):
        wid = lax.axis_index("s") * 2 + lax.axis_index("c")
        row0 = wid * rpt
        bb = row0 // (nblk * rblk)
        jb = (row0 % (nblk * rblk)) // rblk
        rb0 = row0 % rblk
        lane = lax.iota(jnp.int32, 16)
        sems = (s0, s1)
        bufs = (buf0, buf1)

        def dma(blk, par):
            src = score_hbm.at[bb, jb, pl.ds(rb0 + blk * rpb, rpb), :]
            return pltpu.make_async_copy(src, bufs[par], sems[par])

        dma(0, 0).start()
        dma(1, 1).start()

        def do_row(r, blk, par):
            buf = bufs[par]
            pairs = []
            for t in range(8):
                cm = buf[r, pl.ds(16 * t, 16)]
                for j in range(1, 16):
                    cm = jnp.maximum(
                        cm, buf[r, pl.ds(128 * j + 16 * t, 16)])
                pairs.append(plsc.sort_key_val(cm, lane + 16 * t))
            while len(pairs) > 1:
                pairs = [_merge16(*pairs[i], *pairs[i + 1])
                         for i in range(0, len(pairs), 2)]
            ci = pairs[0][1]
            rvec = jnp.full((16,), r, jnp.int32)
            cands = []
            for j in range(16):
                g = plsc.load_gather(buf, [rvec, ci + 128 * j])
                cands.append(plsc.sort_key_val(g, ci + 128 * j))
            while len(cands) > 1:
                cands = [_merge16(*cands[i], *cands[i + 1])
                         for i in range(0, len(cands), 2)]
            gidx = cands[0][1]
            idxout[pl.ds((blk * rpb + r) * KNN, 16)] = gidx

        def do_block(sb, _):
            for par in range(2):
                blk = sb * 2 + par
                dma(blk, par).wait()

                def row2(r2, c):
                    do_row(r2 * 2, blk, par)
                    do_row(r2 * 2 + 1, blk, par)
                    return c
                lax.fori_loop(0, rpb // 2, row2, 0)
                nxt = blk + 2

                @pl.when(nxt < nblks)
                def _():
                    dma(nxt, par).start()
            return 0

        lax.fori_loop(0, nblks // 2, do_block, 0)
        pltpu.sync_copy(idxout, idx_hbm.at[pl.ds(row0 * KNN, rpt * KNN)])

    return sc_topk(score)


def _lin_body(x_ref, w_ref, y1_ref, y2_ref):
    xb = x_ref[0]
    wb = w_ref[...]
    c = xb.shape[0]
    n = xb.shape[1]
    n = xb.shape[1]
    q = n // 4
    w1 = wb[:, :c]
    w2m1 = wb[:, c:] - w1
    dn = (((0,), (1,)), ((), ()))
    y1 = lax.dot_general(xb, w1, dn, preferred_element_type=jnp.float32)
    y2 = lax.dot_general(xb, w2m1, dn, preferred_element_type=jnp.float32)
    for s in range(4):
        y1_ref[0, 0, :, pl.ds(s * GW, GW)] = y1[s * q:(s + 1) * q, :]
        y2_ref[0, 0, :, pl.ds(s * GW, GW)] = y2[s * q:(s + 1) * q, :]


def _lin_call(x, w):
    b, c, n = x.shape
    q = n // 4
    return pl.pallas_call(
        _lin_body,
        grid=(b, CG),
        in_specs=[
            pl.BlockSpec((1, c, n), lambda i, j: (i, 0, 0)),
            pl.BlockSpec((GW, 2 * c), lambda i, j: (j, 0)),
        ],
        out_specs=[
            pl.BlockSpec((1, 1, q, 128), lambda i, j: (i, j, 0, 0)),
            pl.BlockSpec((1, 1, q, 128), lambda i, j: (i, j, 0, 0)),
        ],
        out_shape=[
            jax.ShapeDtypeStruct((b, CG, q, 128), jnp.float32),
            jax.ShapeDtypeStruct((b, CG, q, 128), jnp.float32),
        ],
    )(x, w)


def _bcast_lane(v, k):
    dn = lax.GatherDimensionNumbers(
        offset_dims=(), collapsed_slice_dims=(0,), start_index_map=(0,))
    return lax.gather(v, jnp.full((16, 1), k, jnp.int32), dn, (1,),
                      mode=lax.GatherScatterMode.PROMISE_IN_BOUNDS)


def _gather_call(y1w, idxf, b, n):
    fw = n * GW
    ow = fw // NG
    jn = n // 4 // NG
    nq = n // 4
    y1 = y1w.reshape(b, CG, fw)
    mesh = plsc.VectorSubcoreMesh(core_axis_name="c", subcore_axis_name="s")

    @functools.partial(
        pl.kernel,
        mesh=mesh,
        compiler_params=pltpu.CompilerParams(needs_layout_passes=False),
        out_type=[
            jax.ShapeDtypeStruct((b, CG, nq, 128), jnp.float32),
            jax.ShapeDtypeStruct((b, CG, nq, 128), jnp.float32),
            jax.ShapeDtypeStruct((b, CG, nq, 128), jnp.float32),
        ],
        scratch_types=[
            pltpu.VMEM((fw,), jnp.float32),
            pltpu.VMEM((NG * jn * KNN,), jnp.int32),
            pltpu.VMEM((jn, 128), jnp.float32),
            pltpu.VMEM((jn, 128), jnp.float32),
            pltpu.VMEM((jn, 128), jnp.float32),
            pltpu.SemaphoreType.DMA,
            pltpu.SemaphoreType.DMA,
        ],
    )
    def sc_gather(y1_hbm, idx_hbm, outm_hbm, outs_hbm, outq_hbm,
                  y1v, idxv, mv, sv, qv, sin, sout):
        wid = lax.axis_index("s") * 2 + lax.axis_index("c")
        g = wid // NG
        ng = wid % NG
        lane = lax.iota(jnp.int32, 16)

        def outdrain(bp):
            pltpu.make_async_copy(
                mv, outm_hbm.at[bp, g, pl.ds(ng * jn, jn), :], sout).wait()
            pltpu.make_async_copy(
                sv, outs_hbm.at[bp, g, pl.ds(ng * jn, jn), :], sout).wait()
            pltpu.make_async_copy(
                qv, outq_hbm.at[bp, g, pl.ds(ng * jn, jn), :], sout).wait()

        def idma(bi, q4):
            return pltpu.make_async_copy(
                idx_hbm.at[pl.ds((bi * n + q4 * nq + ng * jn) * KNN,
                                 jn * KNN)],
                idxv.at[pl.ds(q4 * jn * KNN, jn * KNN)], sin)

        for bi in range(b):
            for q4 in range(4):
                idma(bi, q4).start()
            pltpu.sync_copy(y1_hbm.at[bi, g], y1v)
            for q4 in range(4):
                idma(bi, q4).wait()
            if bi > 0:
                outdrain(bi - 1)

            for q4 in range(4):
                def n_body(j, _, q4=q4):
                    irow = idxv[pl.ds((q4 * jn + j) * KNN, 16)]
                    av = ((irow & 511) << 7) + ((irow >> 9) << 5)
                    vm0 = jnp.full((16,), -jnp.inf, jnp.float32)
                    vm1 = vm0
                    vs0 = jnp.zeros((16,), jnp.float32)
                    vs1 = vs0
                    vq0 = vs0
                    vq1 = vs0
                    for k in range(KNN):
                        base = _bcast_lane(av, k) + lane
                        g0 = plsc.load_gather(y1v, [base])
                        g1 = plsc.load_gather(y1v, [base + 16])
                        vm0 = jnp.maximum(vm0, g0)
                        vm1 = jnp.maximum(vm1, g1)
                        vs0 = vs0 + g0
                        vs1 = vs1 + g1
                        vq0 = vq0 + g0 * g0
                        vq1 = vq1 + g1 * g1
                    mv[j, pl.ds(q4 * GW, 16)] = vm0
                    mv[j, pl.ds(q4 * GW + 16, 16)] = vm1
                    sv[j, pl.ds(q4 * GW, 16)] = vs0
                    sv[j, pl.ds(q4 * GW + 16, 16)] = vs1
                    qv[j, pl.ds(q4 * GW, 16)] = vq0
                    qv[j, pl.ds(q4 * GW + 16, 16)] = vq1
                    return 0

                def n_body2(j2, c, q4=q4):
                    n_body(j2 * 2, c, q4=q4)
                    n_body(j2 * 2 + 1, c, q4=q4)
                    return c

                lax.fori_loop(0, jn // 2, n_body2, 0)
            pltpu.make_async_copy(
                mv, outm_hbm.at[bi, g, pl.ds(ng * jn, jn), :], sout).start()
            pltpu.make_async_copy(
                sv, outs_hbm.at[bi, g, pl.ds(ng * jn, jn), :], sout).start()
            pltpu.make_async_copy(
                qv, outq_hbm.at[bi, g, pl.ds(ng * jn, jn), :], sout).start()
        outdrain(b - 1)

    return sc_gather(y1, idxf)


def _fold4(x):
    return (x[:, 0:GW] + x[:, GW:2 * GW] + x[:, 2 * GW:3 * GW]
            + x[:, 3 * GW:4 * GW])


def _stats_body(y2_ref, s1_ref, q1_ref, part_ref):
    y2b = y2_ref[0, 0]
    s1b = s1_ref[0, 0]
    q1b = q1_ref[0, 0]
    g1 = _fold4(jnp.sum(s1b, axis=0, keepdims=True))
    g2 = _fold4(jnp.sum(q1b, axis=0, keepdims=True))
    g3 = _fold4(jnp.sum(y2b * s1b, axis=0, keepdims=True))
    h1 = _fold4(jnp.sum(y2b, axis=0, keepdims=True))
    h2 = _fold4(jnp.sum(y2b * y2b, axis=0, keepdims=True))
    part_ref[0, 0] = jnp.concatenate([g1, g2, g3, h1, h2], axis=0)


def _stats_call(y2, s1, q1):
    b, cg, nq, _ = y2.shape
    return pl.pallas_call(
        _stats_body,
        grid=(b, cg),
        in_specs=[
            pl.BlockSpec((1, 1, nq, 128), lambda i, j: (i, j, 0, 0)),
            pl.BlockSpec((1, 1, nq, 128), lambda i, j: (i, j, 0, 0)),
            pl.BlockSpec((1, 1, nq, 128), lambda i, j: (i, j, 0, 0)),
        ],
        out_specs=pl.BlockSpec((1, 1, 5, GW), lambda i, j: (i, j, 0, 0)),
        out_shape=jax.ShapeDtypeStruct((b, cg, 5, GW), jnp.float32),
    )(y2, s1, q1)


def _final_body(m1_ref, y2_ref, part_ref, gam_ref, bet_ref, out_ref, *, m):
    sums = jnp.sum(part_ref[:, 0], axis=0)
    g1 = sums[0:1]
    g2 = sums[1:2]
    g3 = sums[2:3]
    h1 = sums[3:4]
    h2 = sums[4:5]
    kf = float(KNN)
    mean = (g1 + kf * h1) / m
    e2 = (g2 + 2.0 * g3 + kf * h2) / m
    var = e2 - mean * mean
    inv = lax.rsqrt(var + 1e-5)
    scale = gam_ref[0] * inv
    shift = bet_ref[0] - mean * scale
    scale4 = jnp.concatenate([scale] * 4, axis=1)
    shift4 = jnp.concatenate([shift] * 4, axis=1)
    v = m1_ref[0, 0] + y2_ref[0, 0]
    z = jnp.maximum(v * scale4 + shift4, 0.0)
    nq = z.shape[0]
    for q in range(4):
        out_ref[0, 0, :, pl.ds(q * nq, nq)] = z[:, q * GW:(q + 1) * GW].T


def _final_call(m1, y2, parts, gamma, beta):
    b, cg, nq, _ = m1.shape
    n = nq * 4
    m = float(b * n * KNN)
    return pl.pallas_call(
        functools.partial(_final_body, m=m),
        grid=(b, cg),
        in_specs=[
            pl.BlockSpec((1, 1, nq, 128), lambda i, j: (i, j, 0, 0)),
            pl.BlockSpec((1, 1, nq, 128), lambda i, j: (i, j, 0, 0)),
            pl.BlockSpec((b, 1, 5, GW), lambda i, j: (0, j, 0, 0)),
            pl.BlockSpec((1, 1, GW), lambda i, j: (j, 0, 0)),
            pl.BlockSpec((1, 1, GW), lambda i, j: (j, 0, 0)),
        ],
        out_specs=pl.BlockSpec((1, 1, GW, n), lambda i, j: (i, j, 0, 0)),
        out_shape=jax.ShapeDtypeStruct((b, cg, GW, n), jnp.float32),
    )(m1, y2, parts, gamma, beta)


def kernel(x, W, gamma, beta):
    b, c, n = x.shape
    o = W.shape[0]
    score = _score_call(x)
    y1, y2 = _lin_call(x, W)
    idxf = _topk_call(score)
    m1, s1, q1 = _gather_call(y1, idxf, b, n)
    parts = _stats_call(y2, s1, q1)
    out = _final_call(m1, y2, parts,
                      gamma.reshape(CG, 1, GW), beta.reshape(CG, 1, GW))
    return out.reshape(b, o, n)
